# Initial kernel scaffold; baseline (speedup 1.0000x reference)
import jax
import jax.numpy as jnp
from jax.experimental import pallas as pl


def kernel(node_type_probs, edge_type_probs, w, node):
    raise NotImplementedError("write your pallas kernel here")


# speedup vs baseline: 5.5359x; 5.5359x over previous
import functools

import jax
import jax.numpy as jnp
from jax import lax
from jax.experimental import pallas as pl
from jax.experimental.pallas import tpu as pltpu
from jax.experimental.pallas import tpu_sc as plsc

EPS = 1e-10
L = 16
NC = 2
NS = 16
NW = NC * NS


def _uniform01(x):
    x = x ^ lax.shift_right_logical(x, 16)
    x = x * jnp.int32(-2048144789)
    x = x ^ lax.shift_right_logical(x, 13)
    x = x * jnp.int32(-1028477387)
    x = x ^ lax.shift_right_logical(x, 16)
    bits = lax.shift_right_logical(x, 9) | jnp.int32(0x3F800000)
    return plsc.bitcast(bits, jnp.float32) - jnp.float32(1.0)


def _make_sc_call(n, et):
    assert et == L and n % NW == 0
    rpt = n // NW
    nchunk = (rpt + L - 1) // L
    mesh = plsc.VectorSubcoreMesh(
        core_axis_name="c", subcore_axis_name="s", num_cores=NC,
        num_subcores=NS)

    @functools.partial(
        pl.kernel,
        out_type=(
            jax.ShapeDtypeStruct((NW, L), jnp.float32),
            jax.ShapeDtypeStruct((L,), jnp.float32),
            jax.ShapeDtypeStruct((L,), jnp.float32),
        ),
        mesh=mesh,
        compiler_params=pltpu.CompilerParams(needs_layout_passes=False),
        scratch_types=[
            pltpu.VMEM((rpt * L,), jnp.float32),
            pltpu.VMEM((L * L,), jnp.float32),
            pltpu.VMEM((2 * L,), jnp.float32),
            pltpu.VMEM((2 * L,), jnp.float32),
            pltpu.VMEM((L,), jnp.float32),
            pltpu.VMEM((L,), jnp.int32),
            pltpu.VMEM((L,), jnp.float32),
            pltpu.VMEM((L,), jnp.float32),
            pltpu.SemaphoreType.DMA,
        ],
    )
    def sc_call(edge_hbm, ntp_hbm, nidx_hbm, w_hbm,
                parts_hbm, pnode_hbm, wnode_hbm,
                rows_v, first_v, table_v, ntp_v, red_v, nidx_v, wg_v,
                scal_v, sem):
        wid = lax.axis_index("s") * NC + lax.axis_index("c")
        base = wid * rpt
        pltpu.sync_copy(edge_hbm.at[pl.ds(base * L, rpt * L)], rows_v)
        pltpu.sync_copy(edge_hbm.at[pl.ds(0, L * L)], first_v)

        lanes = lax.iota(jnp.int32, L)
        ones_f = jnp.ones((L,), jnp.float32)
        ones_i = jnp.ones((L,), jnp.int32)
        zeros_i = jnp.zeros((L,), jnp.int32)

        colsum = jnp.zeros((L,), jnp.float32)
        colprod = ones_f
        row_base16 = lanes * L
        for j in range(L):
            c = plsc.load_gather(first_v, [row_base16 + j])
            colsum = colsum + c
            colprod = colprod * c
        rinv = jnp.float32(1.0) / (colsum + EPS)
        r2 = rinv * rinv
        r4 = r2 * r2
        r8 = r4 * r4
        table_v[pl.ds(0, L)] = colprod * (r8 * r8)
        table_v[pl.ds(L, L)] = ones_f

        def chunk(i, acc):
            ridx = i * L + lanes
            valid = ridx < rpt
            ridx_c = jnp.minimum(ridx, rpt - 1)
            u = _uniform01(base + ridx)
            p = jnp.zeros((L,), jnp.float32)
            prefs = []
            flat_base = ridx_c * L
            for j in range(L):
                c = plsc.load_gather(rows_v, [flat_base + j])
                p = p + c
                prefs.append(p)
            t = u * p
            cnt = zeros_i
            for pj in prefs:
                cnt = cnt + jnp.where(pj < t, ones_i, zeros_i)
            vals = plsc.load_gather(table_v, [cnt])
            vals = jnp.where(valid, vals, ones_f)
            return acc * vals

        acc = lax.fori_loop(0, nchunk, chunk, ones_f)

        for s in (8, 4, 2, 1):
            red_v[...] = acc
            acc = acc * plsc.load_gather(red_v, [lanes ^ s])
        red_v[...] = acc
        pltpu.sync_copy(red_v, parts_hbm.at[wid])

        @pl.when(wid == 0)
        def _():
            pltpu.sync_copy(ntp_hbm, ntp_v)
            a = ntp_v[pl.ds(0, L)]
            b = ntp_v[pl.ds(L, L)]
            sa = jnp.sum(a)
            s_tot = sa + jnp.sum(b)
            u = _uniform01(jnp.full((L,), n + 12345, jnp.int32))
            t = u * jnp.broadcast_to(s_tot, (L,))
            ca = plsc.cumsum(a)
            cb = plsc.cumsum(b) + jnp.broadcast_to(sa, (L,))
            cnt = (jnp.sum(jnp.where(ca < t, ones_i, zeros_i))
                   + jnp.sum(jnp.where(cb < t, ones_i, zeros_i)))
            idx = jnp.minimum(jnp.broadcast_to(cnt, (L,)), 2 * L - 1)
            sel = plsc.load_gather(ntp_v, [idx])
            scal_v[...] = sel / (jnp.broadcast_to(s_tot, (L,)) + EPS)
            pltpu.sync_copy(scal_v, pnode_hbm)
            pltpu.sync_copy(nidx_hbm, nidx_v)
            pltpu.async_copy(w_hbm.at[nidx_v], wg_v, sem).wait()
            pltpu.sync_copy(wg_v, wnode_hbm)

    return sc_call


def kernel(node_type_probs, edge_type_probs, w, node):
    n, et = edge_type_probs.shape
    node_idx = jnp.full((L,), node, jnp.int32)
    parts, pnode, wnode = _make_sc_call(n, et)(
        edge_type_probs.reshape(-1), node_type_probs, node_idx, w)
    p_edges = jnp.prod(parts)
    p_O_v = p_edges * pnode[0] + EPS
    n_i = jnp.float32(n)
    traj_len_minus_1 = jnp.float32(n)
    return n_i / traj_len_minus_1 * jnp.log(p_O_v) * wnode[0] / jnp.float32(4)
